# baseline (speedup 1.0000x reference)
import functools

import jax
import jax.numpy as jnp
from jax import lax
from jax.experimental import pallas as pl
from jax.experimental.pallas import tpu as pltpu
from jax.experimental.pallas import tpu_sc as plsc

F32 = jnp.float32
BF16 = jnp.bfloat16

CH = 80
K = 5
CHS = 80
NSUB = 16
NSPLIT = 2

NB_NODE = 2000
EB = 3200


def _pack_bf16_pair(a, p):
    ab = jax.lax.bitcast_convert_type(a, jnp.uint32)
    pb = jax.lax.bitcast_convert_type(p, jnp.uint32)
    ar = (ab + jnp.uint32(0x7FFF) + ((ab >> 16) & jnp.uint32(1))) \
        & jnp.uint32(0xFFFF0000)
    pr = (pb + jnp.uint32(0x7FFF) + ((pb >> 16) & jnp.uint32(1))) >> 16
    return jax.lax.bitcast_convert_type(ar | pr, jnp.int32)


def _unpack_bf16_pair(g):
    gb = jax.lax.bitcast_convert_type(g, jnp.uint32)
    a = jax.lax.bitcast_convert_type(gb & jnp.uint32(0xFFFF0000), F32)
    p = jax.lax.bitcast_convert_type(gb << 16, F32)
    return a, p


def _ln(z, g, c):
    m = jnp.mean(z, axis=-1, keepdims=True)
    v = jnp.mean((z - m) ** 2, axis=-1, keepdims=True)
    return (z - m) * lax.rsqrt(v + 1e-5) * g + c


def _init_body(x_ref, win_ref, bin_ref, wx_ref, bp_ref, hn_ref, cx_ref):
    x = x_ref[...]
    hn_ref[...] = jnp.tanh(
        jnp.dot(x, win_ref[...], preferred_element_type=F32) + bin_ref[...])
    cx_ref[...] = jnp.dot(x, wx_ref[...], preferred_element_type=F32) + bp_ref[...]


def _pre_body(hn_ref, wh_ref, cx_ref, s_ref, e_ref, r_ref):
    ser = jnp.dot(hn_ref[...], wh_ref[...], preferred_element_type=F32) + cx_ref[...]
    s_ref[...] = _pack_bf16_pair(ser[:, 0:128], ser[:, 128:256])
    e_ref[...] = _pack_bf16_pair(ser[:, 256:384], ser[:, 384:512])
    r_ref[...] = ser[:, 512:640]


def _prefin_body(hn_ref, wf_ref, cx_ref, t_ref):
    ab = jnp.dot(hn_ref[...], wf_ref[...], preferred_element_type=F32)
    t_ref[...] = _pack_bf16_pair(ab[:, 0:128] + cx_ref[:, 0:128],
                                 ab[:, 128:256] + cx_ref[:, 256:384])


def _edge_mlp(gs_a, ge_b, p):
    (g1, c1, w2, b2, g2, c2, w3, b3, g3, c3, w4) = p
    z = gs_a.astype(F32) + ge_b.astype(F32)
    h1 = jnp.tanh(_ln(z, g1, c1))
    h2 = jnp.tanh(_ln(jnp.dot(h1.astype(BF16), w2,
                              preferred_element_type=F32) + b2, g2, c2))
    h3 = jnp.tanh(_ln(jnp.dot(h2.astype(BF16), w3,
                              preferred_element_type=F32) + b3, g3, c3))
    return h3, jnp.sum(h3 * w4, axis=-1, keepdims=True)


def _edge_body(gs_ref, ge_ref, g1_ref, c1_ref, w2_ref, b2_ref, g2_ref, c2_ref,
               w3_ref, b3_ref, g3_ref, c3_ref, w4_ref, b4_ref,
               u_ref, v_ref):
    a, pfeat = _unpack_bf16_pair(gs_ref[...])
    b, qfeat = _unpack_bf16_pair(ge_ref[...])
    p = (g1_ref[...], c1_ref[...], w2_ref[...], b2_ref[...], g2_ref[...],
         c2_ref[...], w3_ref[...], b3_ref[...], g3_ref[...], c3_ref[...],
         w4_ref[...])
    _, logit = _edge_mlp(a, b, p)
    e = jax.nn.sigmoid(logit + b4_ref[0])
    u_ref[...] = e * pfeat
    v_ref[...] = e * qfeat


def _edge_fin_body(ga_ref, gb_ref, g1_ref, c1_ref, w2_ref, b2_ref, g2_ref,
                   c2_ref, w3_ref, b3_ref, g3_ref, c3_ref, w4_ref, b4_ref,
                   e_ref):
    p = (g1_ref[...], c1_ref[...], w2_ref[...], b2_ref[...], g2_ref[...],
         c2_ref[...], w3_ref[...], b3_ref[...], g3_ref[...], c3_ref[...],
         w4_ref[...])
    a, _ = _unpack_bf16_pair(ga_ref[...])
    _, b = _unpack_bf16_pair(gb_ref[...])
    _, logit = _edge_mlp(a, b, p)
    e = jax.nn.sigmoid(logit + b4_ref[0])
    e_ref[...] = e.reshape(1, 1, e.shape[0])


def _node_body(mi_ref, mi1_ref, mo_ref, mo1_ref, r_ref, g1_ref, c1_ref,
               w2_ref, b2_ref, g2_ref, c2_ref, w3_ref, b3_ref, g3_ref, c3_ref,
               w4_ref, b4_ref, g4_ref, c4_ref, hn_ref):
    z = ((mi_ref[...] + mi1_ref[...]) + (mo_ref[...] + mo1_ref[...])
         + r_ref[...])
    h1 = jnp.tanh(_ln(z, g1_ref[...], c1_ref[...]))
    h2 = jnp.tanh(_ln(jnp.dot(h1, w2_ref[...], preferred_element_type=F32)
                      + b2_ref[...], g2_ref[...], c2_ref[...]))
    h3 = jnp.tanh(_ln(jnp.dot(h2, w3_ref[...], preferred_element_type=F32)
                      + b3_ref[...], g3_ref[...], c3_ref[...]))
    h4 = jnp.tanh(_ln(jnp.dot(h3, w4_ref[...], preferred_element_type=F32)
                      + b4_ref[...], g4_ref[...], c4_ref[...]))
    hn_ref[...] = h4


def _full(shape):
    nd = len(shape)
    return pl.BlockSpec(shape, lambda i, _n=nd: (0,) * _n)


def _rows(block, ncols):
    return pl.BlockSpec((block, ncols), lambda i: (i, 0))


def _make_gather(n_rows, d, ne, dt):
    per_worker = ne // NSUB
    n_outer = per_worker // (CH * K)
    chunk = K * CH
    mesh = plsc.VectorSubcoreMesh(core_axis_name="c", subcore_axis_name="s")

    @functools.partial(
        pl.kernel, mesh=mesh,
        out_type=[jax.ShapeDtypeStruct((ne, d), dt),
                  jax.ShapeDtypeStruct((ne, d), dt)],
        scratch_types=[pltpu.VMEM((n_outer, K, CH), jnp.int32),
                       pltpu.VMEM((2, K * CH, d), dt),
                       pltpu.SemaphoreType.DMA, pltpu.SemaphoreType.DMA,
                       pltpu.SemaphoreType.DMA, pltpu.SemaphoreType.DMA,
                       pltpu.SemaphoreType.DMA],
    )
    def gk(s_hbm, e_hbm, sidx_hbm, eidx_hbm, gs_hbm, ge_hbm, idxa, buf_v,
           isem, gsem0, gsem1, wsem0, wsem1):
        cid = lax.axis_index("c")
        sid = lax.axis_index("s")
        gsem = (gsem0, gsem1)
        wsem = (wsem0, wsem1)

        def do(src_hbm, idx3_hbm, out_hbm):
            pltpu.async_copy(idx3_hbm.at[pl.ds(sid * n_outer, n_outer)],
                             idxa, isem).wait()
            shape_dst = out_hbm.at[pl.ds(0, chunk)]

            def fetch(c, b):
                @pl.when(c >= 2)
                def _():
                    pltpu.make_async_copy(shape_dst, buf_v.at[b],
                                          wsem[b]).wait()
                for j in range(K):
                    pltpu.async_copy(src_hbm.at[idxa.at[c, j]],
                                     buf_v.at[b].at[pl.ds(j * CH, CH)],
                                     gsem[b])

            def flush(c, b):
                pltpu.make_async_copy(shape_dst, buf_v.at[b], gsem[b]).wait()
                row0 = sid * per_worker + c * chunk
                pltpu.async_copy(buf_v.at[b], out_hbm.at[pl.ds(row0, chunk)],
                                 wsem[b])

            fetch(0, 0)

            @pl.loop(0, n_outer - 1)
            def _(c):
                parity = lax.rem(c, 2)

                @pl.when(parity == 0)
                def _():
                    fetch(c + 1, 1)
                    flush(c, 0)

                @pl.when(parity == 1)
                def _():
                    fetch(c + 1, 0)
                    flush(c, 1)

            flush(n_outer - 1, (n_outer - 1) % 2)
            pltpu.make_async_copy(shape_dst, buf_v.at[0], wsem[0]).wait()
            pltpu.make_async_copy(shape_dst, buf_v.at[1], wsem[1]).wait()

        @pl.when(cid == 0)
        def _():
            do(s_hbm, sidx_hbm, gs_hbm)

        @pl.when(cid == 1)
        def _():
            do(e_hbm, eidx_hbm, ge_hbm)

    return gk


def _make_scatter(n_nodes, ne):
    per_worker = ne // NSUB
    n_outer = per_worker // CHS
    n_writers = 10
    out_rows = n_nodes // n_writers
    mesh = plsc.VectorSubcoreMesh(core_axis_name="c", subcore_axis_name="s")

    nslot = 3

    @functools.partial(
        pl.kernel, mesh=mesh,
        out_type=[jax.ShapeDtypeStruct((n_nodes, 128), F32),
                  jax.ShapeDtypeStruct((n_nodes, 128), F32)],
        scratch_types=[pltpu.VMEM_SHARED((n_nodes, 128), F32),
                       pltpu.VMEM((n_outer, 1, CHS), jnp.int32),
                       pltpu.VMEM((nslot, CHS, 128), F32)]
                      + [pltpu.SemaphoreType.DMA] * (2 * nslot),
    )
    def sk(u_hbm, v_hbm, eidx_hbm, sidx_hbm, zero_hbm, mi_hbm, mo_hbm,
           acc_sh, idxa, pay_v, *sems):
        cid = lax.axis_index("c")
        sid = lax.axis_index("s")
        psem = sems[:nslot]
        ssem = sems[nslot:]

        @pl.when(sid == 0)
        def _():
            pltpu.sync_copy(zero_hbm, acc_sh)
        plsc.subcore_barrier()

        def do(pay_hbm, idx3_hbm):
            pltpu.async_copy(idx3_hbm.at[pl.ds(sid * n_outer, n_outer)],
                             idxa, psem[0]).wait()
            shape_src = pay_hbm.at[pl.ds(0, CHS)]

            def fetch(c, b):
                row0 = sid * per_worker + c * CHS
                pltpu.async_copy(pay_hbm.at[pl.ds(row0, CHS)],
                                 pay_v.at[b], psem[b])

            def stream(c, b):
                pltpu.make_async_copy(shape_src, pay_v.at[b], psem[b]).wait()
                pltpu.async_copy(pay_v.at[b], acc_sh.at[idxa.at[c, 0]],
                                 ssem[b], add=True)

            fetch(0, 0)

            @pl.loop(1, n_outer)
            def _(c):
                parity = lax.rem(c, nslot)
                for b in range(nslot):
                    pb = (b - 1) % nslot

                    @pl.when(parity == b)
                    def _(b=b, pb=pb):
                        @pl.when(c >= nslot)
                        def _():
                            pltpu.make_async_copy(shape_src, pay_v.at[b],
                                                  ssem[b]).wait()
                        fetch(c, b)
                        stream(c - 1, pb)

            stream(n_outer - 1, (n_outer - 1) % nslot)
            for b in range(nslot):
                pltpu.make_async_copy(shape_src, pay_v.at[b], ssem[b]).wait()

        @pl.when(cid == 0)
        def _():
            do(u_hbm, eidx_hbm)

        @pl.when(cid == 1)
        def _():
            do(v_hbm, sidx_hbm)

        plsc.subcore_barrier()
        r0 = sid * out_rows

        @pl.when(jnp.logical_and(cid == 0, sid < n_writers))
        def _():
            pltpu.sync_copy(acc_sh.at[pl.ds(r0, out_rows)],
                            mi_hbm.at[pl.ds(r0, out_rows)])

        @pl.when(jnp.logical_and(cid == 1, sid < n_writers))
        def _():
            pltpu.sync_copy(acc_sh.at[pl.ds(r0, out_rows)],
                            mo_hbm.at[pl.ds(r0, out_rows)])

    return sk


def kernel(x, edge_index, W_in, b_in, edge_params, node_params):
    ep, npar = edge_params, node_params
    n, d_in = x.shape
    n_edges = edge_index.shape[1]
    h = 128

    ne_p = n_edges // NSPLIT
    start = edge_index[0].astype(jnp.int32)
    end = edge_index[1].astype(jnp.int32)
    gch, sch = K * CH, CHS
    s3g = [start[p * ne_p:(p + 1) * ne_p].reshape(ne_p // gch, K, CH)
           for p in range(NSPLIT)]
    e3g = [end[p * ne_p:(p + 1) * ne_p].reshape(ne_p // gch, K, CH)
           for p in range(NSPLIT)]
    s3s = [start[p * ne_p:(p + 1) * ne_p].reshape(ne_p // sch, 1, CHS)
           for p in range(NSPLIT)]
    e3s = [end[p * ne_p:(p + 1) * ne_p].reshape(ne_p // sch, 1, CHS)
           for p in range(NSPLIT)]

    w1e, wn1 = ep['W1'], npar['W1']
    wh = jnp.concatenate([w1e[0:128], wn1[0:128], w1e[256:384],
                          wn1[256:384], wn1[512:640]], axis=1)
    wx = jnp.concatenate([w1e[128:256], wn1[128:256], w1e[384:512],
                          wn1[384:512], wn1[640:768]], axis=1)
    zeros_h = jnp.zeros((h,), F32)
    bp = jnp.concatenate([ep['b1'], zeros_h, zeros_h, zeros_h,
                          npar['b1']]).reshape(1, 640)
    wf = jnp.concatenate([w1e[0:128], w1e[256:384]], axis=1)

    r2 = lambda a: a.reshape(1, h)
    edge_p = [r2(ep['g1']), r2(ep['c1']), ep['W2'].astype(BF16), r2(ep['b2']),
              r2(ep['g2']), r2(ep['c2']), ep['W3'].astype(BF16), r2(ep['b3']),
              r2(ep['g3']), r2(ep['c3']),
              ep['W4'][:, 0].reshape(1, h), ep['b4']]
    node_p = [r2(npar['g1']), r2(npar['c1']), npar['W2'], r2(npar['b2']),
              r2(npar['g2']), r2(npar['c2']), npar['W3'], r2(npar['b3']),
              r2(npar['g3']), r2(npar['c3']), npar['W4'], r2(npar['b4']),
              r2(npar['g4']), r2(npar['c4'])]

    n_nblk = n // NB_NODE
    n_eblk = ne_p // EB

    init_call = pl.pallas_call(
        _init_body, grid=(n_nblk,),
        in_specs=[_rows(NB_NODE, d_in), _full((d_in, h)), _full((1, h)),
                  _full((d_in, 640)), _full((1, 640))],
        out_specs=[_rows(NB_NODE, h), _rows(NB_NODE, 640)],
        out_shape=[jax.ShapeDtypeStruct((n, h), F32),
                   jax.ShapeDtypeStruct((n, 640), F32)],
    )
    hn, cx = init_call(x, W_in, b_in.reshape(1, h), wx, bp)

    pre_call = pl.pallas_call(
        _pre_body, grid=(n_nblk,),
        in_specs=[_rows(NB_NODE, h), _full((h, 640)), _rows(NB_NODE, 640)],
        out_specs=[_rows(NB_NODE, h), _rows(NB_NODE, h),
                   _rows(NB_NODE, h)],
        out_shape=[jax.ShapeDtypeStruct((n, h), jnp.int32),
                   jax.ShapeDtypeStruct((n, h), jnp.int32),
                   jax.ShapeDtypeStruct((n, h), F32)],
    )

    edge_specs = ([_rows(EB, h), _rows(EB, h)]
                  + [_full((1, h)), _full((1, h)), _full((h, h)),
                     _full((1, h)), _full((1, h)), _full((1, h)),
                     _full((h, h)), _full((1, h)), _full((1, h)),
                     _full((1, h)), _full((1, h))]
                  + [pl.BlockSpec(memory_space=pltpu.SMEM)])
    edge_call = pl.pallas_call(
        _edge_body, grid=(n_eblk,),
        in_specs=edge_specs,
        out_specs=[_rows(EB, h), _rows(EB, h)],
        out_shape=[jax.ShapeDtypeStruct((ne_p, h), F32),
                   jax.ShapeDtypeStruct((ne_p, h), F32)],
    )

    node_call = pl.pallas_call(
        _node_body, grid=(n_nblk,),
        in_specs=[_rows(NB_NODE, h)] * 5
                 + [_full((1, h)), _full((1, h)), _full((h, h)),
                    _full((1, h)), _full((1, h)), _full((1, h)),
                    _full((h, h)), _full((1, h)), _full((1, h)),
                    _full((1, h)), _full((h, h)), _full((1, h)),
                    _full((1, h)), _full((1, h))],
        out_specs=[_rows(NB_NODE, h)],
        out_shape=[jax.ShapeDtypeStruct((n, h), F32)],
    )

    gather_pk = _make_gather(n, h, ne_p, jnp.int32)
    scatter = _make_scatter(n, ne_p)
    zeros_acc = jnp.zeros((n, h), F32)

    for _ in range(3):
        s_mat, e_mat, r_mat = pre_call(hn, wh, cx)
        uv = []
        for p in range(NSPLIT):
            gs, ge = gather_pk(s_mat, e_mat, s3g[p], e3g[p])
            uv.append(edge_call(gs, ge, *edge_p))
        ms = [scatter(uv[p][0], uv[p][1], e3s[p], s3s[p], zeros_acc)
              for p in range(NSPLIT)]
        (hn,) = node_call(ms[0][0], ms[1][0], ms[0][1], ms[1][1], r_mat,
                          *node_p)

    prefin_call = pl.pallas_call(
        _prefin_body, grid=(n_nblk,),
        in_specs=[_rows(NB_NODE, h), _full((h, 256)), _rows(NB_NODE, 640)],
        out_specs=[_rows(NB_NODE, h)],
        out_shape=[jax.ShapeDtypeStruct((n, h), jnp.int32)],
    )
    (tf,) = prefin_call(hn, wf, cx)

    edge_fin_call = pl.pallas_call(
        _edge_fin_body, grid=(n_eblk,),
        in_specs=[_rows(EB, h), _rows(EB, h)] + edge_specs[2:],
        out_specs=[pl.BlockSpec((1, 1, EB), lambda i: (i, 0, 0))],
        out_shape=[jax.ShapeDtypeStruct((n_eblk, 1, EB), F32)],
    )
    e_parts = []
    for p in range(NSPLIT):
        ga, gb = gather_pk(tf, tf, s3g[p], e3g[p])
        (e3,) = edge_fin_call(ga, gb, *edge_p)
        e_parts.append(e3.reshape(ne_p))
    return jnp.concatenate(e_parts)


# speedup vs baseline: 1.0001x; 1.0001x over previous
import functools

import jax
import jax.numpy as jnp
from jax import lax
from jax.experimental import pallas as pl
from jax.experimental.pallas import tpu as pltpu
from jax.experimental.pallas import tpu_sc as plsc

F32 = jnp.float32
BF16 = jnp.bfloat16

CH = 80
K = 5
CHS = 80
NSUB = 16
NSPLIT = 2

NB_NODE = 2000
EB = 3200


def _pack_bf16_pair(a, p):
    ab = jax.lax.bitcast_convert_type(a, jnp.uint32)
    pb = jax.lax.bitcast_convert_type(p, jnp.uint32)
    ar = (ab + jnp.uint32(0x7FFF) + ((ab >> 16) & jnp.uint32(1))) \
        & jnp.uint32(0xFFFF0000)
    pr = (pb + jnp.uint32(0x7FFF) + ((pb >> 16) & jnp.uint32(1))) >> 16
    return jax.lax.bitcast_convert_type(ar | pr, jnp.int32)


def _unpack_bf16_pair(g):
    gb = jax.lax.bitcast_convert_type(g, jnp.uint32)
    a = jax.lax.bitcast_convert_type(gb & jnp.uint32(0xFFFF0000), F32)
    p = jax.lax.bitcast_convert_type(gb << 16, F32)
    return a, p


def _ln(z, g, c):
    m = jnp.mean(z, axis=-1, keepdims=True)
    v = jnp.mean((z - m) ** 2, axis=-1, keepdims=True)
    return (z - m) * lax.rsqrt(v + 1e-5) * g + c


def _init_body(x_ref, win_ref, bin_ref, wx_ref, bp_ref, hn_ref, cx_ref):
    x = x_ref[...]
    hn_ref[...] = jnp.tanh(
        jnp.dot(x, win_ref[...], preferred_element_type=F32) + bin_ref[...])
    cx_ref[...] = jnp.dot(x, wx_ref[...], preferred_element_type=F32) + bp_ref[...]


def _pre_body(hn_ref, wh_ref, cx_ref, s_ref, e_ref, r_ref):
    ser = jnp.dot(hn_ref[...], wh_ref[...], preferred_element_type=F32) + cx_ref[...]
    s_ref[...] = _pack_bf16_pair(ser[:, 0:128], ser[:, 128:256])
    e_ref[...] = _pack_bf16_pair(ser[:, 256:384], ser[:, 384:512])
    r_ref[...] = ser[:, 512:640]


def _prefin_body(hn_ref, wf_ref, cx_ref, t_ref):
    ab = jnp.dot(hn_ref[...], wf_ref[...], preferred_element_type=F32)
    t_ref[...] = _pack_bf16_pair(ab[:, 0:128] + cx_ref[:, 0:128],
                                 ab[:, 128:256] + cx_ref[:, 256:384])


def _edge_mlp(gs_a, ge_b, p):
    (g1, c1, w2, b2, g2, c2, w3, b3, g3, c3, w4) = p
    z = gs_a.astype(F32) + ge_b.astype(F32)
    h1 = jnp.tanh(_ln(z, g1, c1))
    h2 = jnp.tanh(_ln(jnp.dot(h1.astype(BF16), w2,
                              preferred_element_type=F32) + b2, g2, c2))
    h3 = jnp.tanh(_ln(jnp.dot(h2.astype(BF16), w3,
                              preferred_element_type=F32) + b3, g3, c3))
    return h3, jnp.sum(h3 * w4, axis=-1, keepdims=True)


def _edge_body(gs_ref, ge_ref, g1_ref, c1_ref, w2_ref, b2_ref, g2_ref, c2_ref,
               w3_ref, b3_ref, g3_ref, c3_ref, w4_ref, b4_ref,
               u_ref, v_ref):
    a, pfeat = _unpack_bf16_pair(gs_ref[...])
    b, qfeat = _unpack_bf16_pair(ge_ref[...])
    p = (g1_ref[...], c1_ref[...], w2_ref[...], b2_ref[...], g2_ref[...],
         c2_ref[...], w3_ref[...], b3_ref[...], g3_ref[...], c3_ref[...],
         w4_ref[...])
    _, logit = _edge_mlp(a, b, p)
    e = jax.nn.sigmoid(logit + b4_ref[0])
    u_ref[...] = e * pfeat
    v_ref[...] = e * qfeat


def _edge_fin_body(ga_ref, gb_ref, g1_ref, c1_ref, w2_ref, b2_ref, g2_ref,
                   c2_ref, w3_ref, b3_ref, g3_ref, c3_ref, w4_ref, b4_ref,
                   e_ref):
    p = (g1_ref[...], c1_ref[...], w2_ref[...], b2_ref[...], g2_ref[...],
         c2_ref[...], w3_ref[...], b3_ref[...], g3_ref[...], c3_ref[...],
         w4_ref[...])
    a, _ = _unpack_bf16_pair(ga_ref[...])
    _, b = _unpack_bf16_pair(gb_ref[...])
    _, logit = _edge_mlp(a, b, p)
    e = jax.nn.sigmoid(logit + b4_ref[0])
    e_ref[...] = e.reshape(1, 1, e.shape[0])


def _node_body(mi_ref, mi1_ref, mo_ref, mo1_ref, r_ref, g1_ref, c1_ref,
               w2_ref, b2_ref, g2_ref, c2_ref, w3_ref, b3_ref, g3_ref, c3_ref,
               w4_ref, b4_ref, g4_ref, c4_ref, hn_ref):
    z = ((mi_ref[...] + mi1_ref[...]) + (mo_ref[...] + mo1_ref[...])
         + r_ref[...])
    h1 = jnp.tanh(_ln(z, g1_ref[...], c1_ref[...]))
    h2 = jnp.tanh(_ln(jnp.dot(h1, w2_ref[...], preferred_element_type=F32)
                      + b2_ref[...], g2_ref[...], c2_ref[...]))
    h3 = jnp.tanh(_ln(jnp.dot(h2, w3_ref[...], preferred_element_type=F32)
                      + b3_ref[...], g3_ref[...], c3_ref[...]))
    h4 = jnp.tanh(_ln(jnp.dot(h3, w4_ref[...], preferred_element_type=F32)
                      + b4_ref[...], g4_ref[...], c4_ref[...]))
    hn_ref[...] = h4


def _full(shape):
    nd = len(shape)
    return pl.BlockSpec(shape, lambda i, _n=nd: (0,) * _n)


def _rows(block, ncols):
    return pl.BlockSpec((block, ncols), lambda i: (i, 0))


def _make_gather(n_rows, d, ne, dt):
    per_worker = ne // NSUB
    n_outer = per_worker // (CH * K)
    chunk = K * CH
    mesh = plsc.VectorSubcoreMesh(core_axis_name="c", subcore_axis_name="s")

    @functools.partial(
        pl.kernel, mesh=mesh,
        out_type=[jax.ShapeDtypeStruct((ne, d), dt),
                  jax.ShapeDtypeStruct((ne, d), dt)],
        scratch_types=[pltpu.VMEM((n_outer, K, CH), jnp.int32),
                       pltpu.VMEM((2, K * CH, d), dt),
                       pltpu.SemaphoreType.DMA, pltpu.SemaphoreType.DMA,
                       pltpu.SemaphoreType.DMA, pltpu.SemaphoreType.DMA,
                       pltpu.SemaphoreType.DMA],
    )
    def gk(s_hbm, e_hbm, sidx_hbm, eidx_hbm, gs_hbm, ge_hbm, idxa, buf_v,
           isem, gsem0, gsem1, wsem0, wsem1):
        cid = lax.axis_index("c")
        sid = lax.axis_index("s")
        gsem = (gsem0, gsem1)
        wsem = (wsem0, wsem1)

        def do(src_hbm, idx3_hbm, out_hbm):
            pltpu.async_copy(idx3_hbm.at[pl.ds(sid * n_outer, n_outer)],
                             idxa, isem).wait()
            shape_dst = out_hbm.at[pl.ds(0, chunk)]

            def fetch(c, b):
                @pl.when(c >= 2)
                def _():
                    pltpu.make_async_copy(shape_dst, buf_v.at[b],
                                          wsem[b]).wait()
                for j in range(K):
                    pltpu.async_copy(src_hbm.at[idxa.at[c, j]],
                                     buf_v.at[b].at[pl.ds(j * CH, CH)],
                                     gsem[b])

            def flush(c, b):
                pltpu.make_async_copy(shape_dst, buf_v.at[b], gsem[b]).wait()
                row0 = sid * per_worker + c * chunk
                pltpu.async_copy(buf_v.at[b], out_hbm.at[pl.ds(row0, chunk)],
                                 wsem[b])

            fetch(0, 0)

            @pl.loop(0, n_outer - 1)
            def _(c):
                parity = lax.rem(c, 2)

                @pl.when(parity == 0)
                def _():
                    fetch(c + 1, 1)
                    flush(c, 0)

                @pl.when(parity == 1)
                def _():
                    fetch(c + 1, 0)
                    flush(c, 1)

            flush(n_outer - 1, (n_outer - 1) % 2)
            pltpu.make_async_copy(shape_dst, buf_v.at[0], wsem[0]).wait()
            pltpu.make_async_copy(shape_dst, buf_v.at[1], wsem[1]).wait()

        @pl.when(cid == 0)
        def _():
            do(s_hbm, sidx_hbm, gs_hbm)

        @pl.when(cid == 1)
        def _():
            do(e_hbm, eidx_hbm, ge_hbm)

    return gk


def _make_scatter(n_nodes, ne):
    per_worker = ne // NSUB
    n_outer = per_worker // CHS
    n_writers = 5
    out_rows = n_nodes // n_writers
    mesh = plsc.VectorSubcoreMesh(core_axis_name="c", subcore_axis_name="s")

    nslot = 3

    @functools.partial(
        pl.kernel, mesh=mesh,
        out_type=[jax.ShapeDtypeStruct((n_nodes, 128), F32),
                  jax.ShapeDtypeStruct((n_nodes, 128), F32)],
        scratch_types=[pltpu.VMEM_SHARED((n_nodes, 128), F32),
                       pltpu.VMEM((n_outer, 1, CHS), jnp.int32),
                       pltpu.VMEM((nslot, CHS, 128), F32)]
                      + [pltpu.SemaphoreType.DMA] * (2 * nslot),
    )
    def sk(u_hbm, v_hbm, eidx_hbm, sidx_hbm, zero_hbm, mi_hbm, mo_hbm,
           acc_sh, idxa, pay_v, *sems):
        cid = lax.axis_index("c")
        sid = lax.axis_index("s")
        psem = sems[:nslot]
        ssem = sems[nslot:]

        @pl.when(sid == 0)
        def _():
            pltpu.sync_copy(zero_hbm, acc_sh)
        plsc.subcore_barrier()

        def do(pay_hbm, idx4_hbm):
            pltpu.async_copy(idx4_hbm.at[sid], idxa, psem[0]).wait()
            shape_src = pay_hbm.at[pl.ds(0, CHS)]

            def fetch(c, b):
                row0 = sid * CHS + c * (NSUB * CHS)
                pltpu.async_copy(pay_hbm.at[pl.ds(row0, CHS)],
                                 pay_v.at[b], psem[b])

            def stream(c, b):
                pltpu.make_async_copy(shape_src, pay_v.at[b], psem[b]).wait()
                pltpu.async_copy(pay_v.at[b], acc_sh.at[idxa.at[c, 0]],
                                 ssem[b], add=True)

            fetch(0, 0)

            @pl.loop(1, n_outer)
            def _(c):
                parity = lax.rem(c, nslot)
                for b in range(nslot):
                    pb = (b - 1) % nslot

                    @pl.when(parity == b)
                    def _(b=b, pb=pb):
                        @pl.when(c >= nslot)
                        def _():
                            pltpu.make_async_copy(shape_src, pay_v.at[b],
                                                  ssem[b]).wait()
                        fetch(c, b)
                        stream(c - 1, pb)

            stream(n_outer - 1, (n_outer - 1) % nslot)
            for b in range(nslot):
                pltpu.make_async_copy(shape_src, pay_v.at[b], ssem[b]).wait()

        @pl.when(cid == 0)
        def _():
            do(u_hbm, eidx_hbm)

        @pl.when(cid == 1)
        def _():
            do(v_hbm, sidx_hbm)

        plsc.subcore_barrier()
        r0 = sid * out_rows

        @pl.when(jnp.logical_and(cid == 0, sid < n_writers))
        def _():
            pltpu.sync_copy(acc_sh.at[pl.ds(r0, out_rows)],
                            mi_hbm.at[pl.ds(r0, out_rows)])

        @pl.when(jnp.logical_and(cid == 1, sid < n_writers))
        def _():
            pltpu.sync_copy(acc_sh.at[pl.ds(r0, out_rows)],
                            mo_hbm.at[pl.ds(r0, out_rows)])

    return sk


def kernel(x, edge_index, W_in, b_in, edge_params, node_params):
    ep, npar = edge_params, node_params
    n, d_in = x.shape
    n_edges = edge_index.shape[1]
    h = 128

    ne_p = n_edges // NSPLIT
    start = edge_index[0].astype(jnp.int32)
    end = edge_index[1].astype(jnp.int32)
    gch, sch = K * CH, CHS
    s3g = [start[p * ne_p:(p + 1) * ne_p].reshape(ne_p // gch, K, CH)
           for p in range(NSPLIT)]
    e3g = [end[p * ne_p:(p + 1) * ne_p].reshape(ne_p // gch, K, CH)
           for p in range(NSPLIT)]
    def _sidx(v, p):
        part = v[p * ne_p:(p + 1) * ne_p]
        return part.reshape(ne_p // (sch * NSUB), NSUB, 1, CHS).transpose(
            1, 0, 2, 3)

    s3s = [_sidx(start, p) for p in range(NSPLIT)]
    e3s = [_sidx(end, p) for p in range(NSPLIT)]

    w1e, wn1 = ep['W1'], npar['W1']
    wh = jnp.concatenate([w1e[0:128], wn1[0:128], w1e[256:384],
                          wn1[256:384], wn1[512:640]], axis=1)
    wx = jnp.concatenate([w1e[128:256], wn1[128:256], w1e[384:512],
                          wn1[384:512], wn1[640:768]], axis=1)
    zeros_h = jnp.zeros((h,), F32)
    bp = jnp.concatenate([ep['b1'], zeros_h, zeros_h, zeros_h,
                          npar['b1']]).reshape(1, 640)
    wf = jnp.concatenate([w1e[0:128], w1e[256:384]], axis=1)

    r2 = lambda a: a.reshape(1, h)
    edge_p = [r2(ep['g1']), r2(ep['c1']), ep['W2'].astype(BF16), r2(ep['b2']),
              r2(ep['g2']), r2(ep['c2']), ep['W3'].astype(BF16), r2(ep['b3']),
              r2(ep['g3']), r2(ep['c3']),
              ep['W4'][:, 0].reshape(1, h), ep['b4']]
    node_p = [r2(npar['g1']), r2(npar['c1']), npar['W2'], r2(npar['b2']),
              r2(npar['g2']), r2(npar['c2']), npar['W3'], r2(npar['b3']),
              r2(npar['g3']), r2(npar['c3']), npar['W4'], r2(npar['b4']),
              r2(npar['g4']), r2(npar['c4'])]

    n_nblk = n // NB_NODE
    n_eblk = ne_p // EB

    init_call = pl.pallas_call(
        _init_body, grid=(n_nblk,),
        in_specs=[_rows(NB_NODE, d_in), _full((d_in, h)), _full((1, h)),
                  _full((d_in, 640)), _full((1, 640))],
        out_specs=[_rows(NB_NODE, h), _rows(NB_NODE, 640)],
        out_shape=[jax.ShapeDtypeStruct((n, h), F32),
                   jax.ShapeDtypeStruct((n, 640), F32)],
    )
    hn, cx = init_call(x, W_in, b_in.reshape(1, h), wx, bp)

    pre_call = pl.pallas_call(
        _pre_body, grid=(n_nblk,),
        in_specs=[_rows(NB_NODE, h), _full((h, 640)), _rows(NB_NODE, 640)],
        out_specs=[_rows(NB_NODE, h), _rows(NB_NODE, h),
                   _rows(NB_NODE, h)],
        out_shape=[jax.ShapeDtypeStruct((n, h), jnp.int32),
                   jax.ShapeDtypeStruct((n, h), jnp.int32),
                   jax.ShapeDtypeStruct((n, h), F32)],
    )

    edge_specs = ([_rows(EB, h), _rows(EB, h)]
                  + [_full((1, h)), _full((1, h)), _full((h, h)),
                     _full((1, h)), _full((1, h)), _full((1, h)),
                     _full((h, h)), _full((1, h)), _full((1, h)),
                     _full((1, h)), _full((1, h))]
                  + [pl.BlockSpec(memory_space=pltpu.SMEM)])
    edge_call = pl.pallas_call(
        _edge_body, grid=(n_eblk,),
        in_specs=edge_specs,
        out_specs=[_rows(EB, h), _rows(EB, h)],
        out_shape=[jax.ShapeDtypeStruct((ne_p, h), F32),
                   jax.ShapeDtypeStruct((ne_p, h), F32)],
    )

    node_call = pl.pallas_call(
        _node_body, grid=(n_nblk,),
        in_specs=[_rows(NB_NODE, h)] * 5
                 + [_full((1, h)), _full((1, h)), _full((h, h)),
                    _full((1, h)), _full((1, h)), _full((1, h)),
                    _full((h, h)), _full((1, h)), _full((1, h)),
                    _full((1, h)), _full((h, h)), _full((1, h)),
                    _full((1, h)), _full((1, h))],
        out_specs=[_rows(NB_NODE, h)],
        out_shape=[jax.ShapeDtypeStruct((n, h), F32)],
    )

    gather_pk = _make_gather(n, h, ne_p, jnp.int32)
    scatter = _make_scatter(n, ne_p)
    zeros_acc = jnp.zeros((n, h), F32)

    for _ in range(3):
        s_mat, e_mat, r_mat = pre_call(hn, wh, cx)
        uv = []
        for p in range(NSPLIT):
            gs, ge = gather_pk(s_mat, e_mat, s3g[p], e3g[p])
            uv.append(edge_call(gs, ge, *edge_p))
        ms = [scatter(uv[p][0], uv[p][1], e3s[p], s3s[p], zeros_acc)
              for p in range(NSPLIT)]
        (hn,) = node_call(ms[0][0], ms[1][0], ms[0][1], ms[1][1], r_mat,
                          *node_p)

    prefin_call = pl.pallas_call(
        _prefin_body, grid=(n_nblk,),
        in_specs=[_rows(NB_NODE, h), _full((h, 256)), _rows(NB_NODE, 640)],
        out_specs=[_rows(NB_NODE, h)],
        out_shape=[jax.ShapeDtypeStruct((n, h), jnp.int32)],
    )
    (tf,) = prefin_call(hn, wf, cx)

    edge_fin_call = pl.pallas_call(
        _edge_fin_body, grid=(n_eblk,),
        in_specs=[_rows(EB, h), _rows(EB, h)] + edge_specs[2:],
        out_specs=[pl.BlockSpec((1, 1, EB), lambda i: (i, 0, 0))],
        out_shape=[jax.ShapeDtypeStruct((n_eblk, 1, EB), F32)],
    )
    e_parts = []
    for p in range(NSPLIT):
        ga, gb = gather_pk(tf, tf, s3g[p], e3g[p])
        (e3,) = edge_fin_call(ga, gb, *edge_p)
        e_parts.append(e3.reshape(ne_p))
    return jnp.concatenate(e_parts)
